# baseline (speedup 1.0000x reference)
import functools

import jax
import jax.numpy as jnp
from jax import lax
from jax.experimental import pallas as pl
from jax.experimental.pallas import tpu as pltpu
from jax.experimental.pallas import tpu_sc as plsc

B_POS = 16384
B_NEG = 81920
B_TOT = B_POS + B_NEG
D = 64
L = 16
IDX_W = 128

NC = 2
NS = 16
NW = NC * NS

ROWS_W = B_TOT // NW
IROWS_W = ROWS_W // IDX_W
CH = 512
N_CH = ROWS_W // CH
CH_IROWS = CH // IDX_W


def _sc_scores(u_idx, v_idx, U, V):
  mesh = plsc.VectorSubcoreMesh(core_axis_name="c", subcore_axis_name="s")

  @functools.partial(
      pl.kernel,
      out_type=jax.ShapeDtypeStruct((B_TOT,), jnp.float32),
      mesh=mesh,
      scratch_types=[
          pltpu.VMEM((IROWS_W, IDX_W), jnp.int32),
          pltpu.VMEM((IROWS_W, IDX_W), jnp.int32),
          pltpu.VMEM((CH, D), jnp.float32),
          pltpu.VMEM((CH, D), jnp.float32),
          pltpu.VMEM((ROWS_W,), jnp.float32),
          pltpu.SemaphoreType.DMA,
      ],
      compiler_params=pltpu.CompilerParams(use_tc_tiling_on_sc=False),
  )
  def k(u_idx_hbm, v_idx_hbm, u_hbm, v_hbm, out_hbm,
        uix, vix, urows, vrows, sc, sem):
    wid = lax.axis_index("s") * NC + lax.axis_index("c")
    ibase = wid * IROWS_W
    pltpu.sync_copy(u_idx_hbm.at[pl.ds(ibase, IROWS_W)], uix)
    pltpu.sync_copy(v_idx_hbm.at[pl.ds(ibase, IROWS_W)], vix)

    for c in range(N_CH):
      dmas = []
      for j in range(CH_IROWS):
        r = c * CH_IROWS + j
        dmas.append(pltpu.async_copy(
            u_hbm.at[uix.at[r]], urows.at[pl.ds(j * IDX_W, IDX_W)], sem))
        dmas.append(pltpu.async_copy(
            v_hbm.at[vix.at[r]], vrows.at[pl.ds(j * IDX_W, IDX_W)], sem))
      for dma in dmas:
        dma.wait()

      lane = lax.iota(jnp.int32, L)
      perms = [lane ^ d for d in (8, 4, 2, 1)]

      @plsc.parallel_loop(0, CH // L, unroll=2)
      def _(g):
        base = g * L
        svec = jnp.zeros((L,), jnp.float32)
        for p in range(L):
          i = base + p
          acc = urows[i, pl.ds(0, L)] * vrows[i, pl.ds(0, L)]
          for d in range(1, D // L):
            acc += urows[i, pl.ds(d * L, L)] * vrows[i, pl.ds(d * L, L)]
          for perm in perms:
            acc = acc + acc.at[perm].get(mode="promise_in_bounds",
                                         unique_indices=True)
          svec = jnp.where(lane == p, acc, svec)
        sc[pl.ds(c * CH + base, L)] = svec

    pltpu.sync_copy(sc, out_hbm.at[pl.ds(wid * ROWS_W, ROWS_W)])

  return k(u_idx, v_idx, U, V)


def _tc_loss(scores):
  x = scores.reshape(B_TOT // 128, 128)
  pos_rows = B_POS // 128

  def body(x_ref, o_ref):
    xv = x_ref[...]
    row = lax.broadcasted_iota(jnp.int32, xv.shape, 0)
    sgn = jnp.where(row < pos_rows, 1.0, -1.0)
    o_ref[0, 0] = -jnp.sum(jax.nn.log_sigmoid(xv * sgn))

  out = pl.pallas_call(
      body,
      out_shape=jax.ShapeDtypeStruct((1, 1), jnp.float32),
      out_specs=pl.BlockSpec(memory_space=pltpu.SMEM),
  )(x)
  return out[0, 0]


@jax.jit
def kernel(pos_u, pos_v, neg_u, neg_v, U, V):
  u_idx = jnp.concatenate([pos_u, neg_u]).astype(jnp.int32).reshape(-1, IDX_W)
  v_idx = jnp.concatenate([pos_v, neg_v]).astype(jnp.int32).reshape(-1, IDX_W)
  scores = _sc_scores(u_idx, v_idx, U, V)
  return _tc_loss(scores)
